# baseline (speedup 1.0000x reference)
import functools

import jax
import jax.numpy as jnp
from jax import lax
from jax.experimental import pallas as pl
from jax.experimental.pallas import tpu as pltpu
from jax.experimental.pallas import tpu_sc as plsc

VOCAB = 1000000
SEQ = 200
BATCH = 4096
D = 32
NC = 2
NS = 16
NW = NC * NS
BPW = BATCH // NW
S = 10
NCHUNK = SEQ // S
LANES = 16


def _emb_avg(text_hbm, table_hbm, out_hbm, idx_v, rows_v, dst_idx, acc_v,
             acc_sh, sem0, sem1):
    cid = lax.axis_index("c")
    sid = lax.axis_index("s")
    wid = cid * NS + sid
    base = wid * BPW
    sh_base = sid * BPW
    sems = (sem0, sem1)

    zero16 = jnp.zeros((LANES,), jnp.float32)
    iota16 = lax.iota(jnp.int32, LANES)

    def fill_dst(i, _):
        def inner(j, _):
            dst_idx[pl.ds(i * BPW + j * LANES, LANES)] = (
                sh_base + j * LANES + iota16)
            return _
        return lax.fori_loop(0, BPW // LANES, inner, _)
    lax.fori_loop(0, S, fill_dst, None)

    def zbody(b, _):
        acc_v[b, pl.ds(0, LANES)] = zero16
        acc_v[b, pl.ds(LANES, LANES)] = zero16
        return _
    lax.fori_loop(0, BPW, zbody, None)
    pltpu.sync_copy(acc_v, acc_sh.at[pl.ds(sh_base, BPW)])

    def fire(c, k):
        pltpu.sync_copy(
            text_hbm.at[pl.ds(c * S, S), pl.ds(base, BPW)], idx_v.at[k])
        for s in range(S):
            pltpu.async_copy(
                table_hbm.at[idx_v.at[k, s]],
                rows_v.at[k, pl.ds(s * BPW, BPW)],
                sems[k],
            )

    def drain_and_accumulate(k):
        pltpu.make_async_copy(
            table_hbm.at[pl.ds(0, S * BPW)], rows_v.at[k], sems[k]).wait()
        pltpu.sync_copy(rows_v.at[k], acc_sh.at[dst_idx], add=True)

    fire(0, 0)

    def pair_body(i, _):
        c0 = 2 * i

        @pl.when(c0 + 1 < NCHUNK)
        def _fire1():
            fire(c0 + 1, 1)

        drain_and_accumulate(0)

        @pl.when(c0 + 2 < NCHUNK)
        def _fire0():
            fire(c0 + 2, 0)

        @pl.when(c0 + 1 < NCHUNK)
        def _drain1():
            drain_and_accumulate(1)
        return _

    lax.fori_loop(0, (NCHUNK + 1) // 2, pair_body, None)

    pltpu.sync_copy(acc_sh.at[pl.ds(sh_base, BPW)], acc_v)
    inv = jnp.float32(1.0 / SEQ)

    def scale_body(b, _):
        acc_v[b, pl.ds(0, LANES)] = acc_v[b, pl.ds(0, LANES)] * inv
        acc_v[b, pl.ds(LANES, LANES)] = acc_v[b, pl.ds(LANES, LANES)] * inv
        return _
    lax.fori_loop(0, BPW, scale_body, None)

    pltpu.sync_copy(acc_v, out_hbm.at[pl.ds(base, BPW)])


PADC = 8192
Q = 262144
NB = Q // PADC
LAST_BLK = (VOCAB - 1) // PADC


def _pack_body(x0_ref, x1_ref, x2_ref, x3_ref, out_ref):
    eye = (lax.broadcasted_iota(jnp.int32, (D, D), 0) ==
           lax.broadcasted_iota(jnp.int32, (D, D), 1)).astype(jnp.float32)

    def t(ref):
        return lax.dot_general(
            ref[...], eye, (((0,), (0,)), ((), ())),
            preferred_element_type=jnp.float32,
            precision=lax.Precision.HIGHEST)

    out_ref[...] = jnp.concatenate(
        [t(x0_ref), t(x1_ref), t(x2_ref), t(x3_ref)], axis=1)


def _pack_table(table_t):
    return pl.pallas_call(
        _pack_body,
        grid=(NB,),
        in_specs=[
            pl.BlockSpec(
                (D, PADC),
                lambda i, k=k: (0, jnp.minimum(i + k * NB, LAST_BLK)))
            for k in range(4)
        ],
        out_specs=pl.BlockSpec((PADC, 4 * D), lambda i: (i, 0)),
        out_shape=jax.ShapeDtypeStruct((Q, 4 * D), jnp.float32),
    )(table_t, table_t, table_t, table_t)


def kernel(text, embedding_weight):
    text = text.astype(jnp.int32)
    packed = _pack_table(jnp.transpose(embedding_weight))
    embedding_weight = jnp.reshape(packed, (4 * Q, D))
    text = 4 * (text & (Q - 1)) + (text >> 18)
    mesh = plsc.VectorSubcoreMesh(core_axis_name="c", subcore_axis_name="s")
    f = functools.partial(
        pl.kernel,
        mesh=mesh,
        compiler_params=pltpu.CompilerParams(use_tc_tiling_on_sc=False),
        out_type=jax.ShapeDtypeStruct((BATCH, D), jnp.float32),
        scratch_types=[
            pltpu.VMEM((2, S, BPW), jnp.int32),
            pltpu.VMEM((2, S * BPW, D), jnp.float32),
            pltpu.VMEM((S * BPW,), jnp.int32),
            pltpu.VMEM((BPW, D), jnp.float32),
            pltpu.VMEM_SHARED((NS * BPW, D), jnp.float32),
            pltpu.SemaphoreType.DMA,
            pltpu.SemaphoreType.DMA,
        ],
    )(_emb_avg)
    return f(text, embedding_weight)


# speedup vs baseline: 3.5981x; 3.5981x over previous
import functools

import jax
import jax.numpy as jnp
from jax import lax
from jax.experimental import pallas as pl
from jax.experimental.pallas import tpu as pltpu
from jax.experimental.pallas import tpu_sc as plsc

VOCAB = 1000000
SEQ = 200
BATCH = 4096
D = 32
NC = 2
NS = 16
NW = NC * NS
BPW = BATCH // NW
S = 10
NCHUNK = SEQ // S
LANES = 16


def _emb_avg(text_hbm, table_hbm, out_hbm, idx_v, rows_v, dst_idx, acc_v,
             acc_sh, sem0, sem1):
    cid = lax.axis_index("c")
    sid = lax.axis_index("s")
    wid = cid * NS + sid
    base = wid * BPW
    sh_base = sid * BPW
    sems = (sem0, sem1)

    zero16 = jnp.zeros((LANES,), jnp.float32)
    iota16 = lax.iota(jnp.int32, LANES)

    def fill_dst(i, _):
        def inner(j, _):
            dst_idx[pl.ds(i * BPW + j * LANES, LANES)] = (
                sh_base + j * LANES + iota16)
            return _
        return lax.fori_loop(0, BPW // LANES, inner, _)
    lax.fori_loop(0, S, fill_dst, None)

    def zbody(b, _):
        acc_v[b, pl.ds(0, LANES)] = zero16
        acc_v[b, pl.ds(LANES, LANES)] = zero16
        return _
    lax.fori_loop(0, BPW, zbody, None)
    pltpu.sync_copy(acc_v, acc_sh.at[pl.ds(sh_base, BPW)])

    def fire(c, k):
        pltpu.sync_copy(
            text_hbm.at[pl.ds(c * S, S), pl.ds(base, BPW)], idx_v.at[k])
        for s in range(S):
            pltpu.async_copy(
                table_hbm.at[idx_v.at[k, s]],
                rows_v.at[k, pl.ds(s * BPW, BPW)],
                sems[k],
            )

    def drain_and_accumulate(k):
        pltpu.make_async_copy(
            table_hbm.at[pl.ds(0, S * BPW)], rows_v.at[k], sems[k]).wait()
        pltpu.sync_copy(rows_v.at[k], acc_sh.at[dst_idx], add=True)

    fire(0, 0)

    def pair_body(i, _):
        c0 = 2 * i

        @pl.when(c0 + 1 < NCHUNK)
        def _fire1():
            fire(c0 + 1, 1)

        drain_and_accumulate(0)

        @pl.when(c0 + 2 < NCHUNK)
        def _fire0():
            fire(c0 + 2, 0)

        @pl.when(c0 + 1 < NCHUNK)
        def _drain1():
            drain_and_accumulate(1)
        return _

    lax.fori_loop(0, (NCHUNK + 1) // 2, pair_body, None)

    pltpu.sync_copy(acc_sh.at[pl.ds(sh_base, BPW)], acc_v)
    inv = jnp.float32(1.0 / SEQ)

    def scale_body(b, _):
        acc_v[b, pl.ds(0, LANES)] = acc_v[b, pl.ds(0, LANES)] * inv
        acc_v[b, pl.ds(LANES, LANES)] = acc_v[b, pl.ds(LANES, LANES)] * inv
        return _
    lax.fori_loop(0, BPW, scale_body, None)

    pltpu.sync_copy(acc_v, out_hbm.at[pl.ds(base, BPW)])


PADC = 8192
Q = 262144
NB = Q // PADC
LAST_BLK = (VOCAB - 1) // PADC


def _pack_body(x0_ref, x1_ref, x2_ref, x3_ref, out_ref):
    out_ref[...] = jnp.transpose(jnp.concatenate(
        [x0_ref[...], x1_ref[...], x2_ref[...], x3_ref[...]], axis=0))


def _pack_table(table_t):
    return pl.pallas_call(
        _pack_body,
        grid=(NB,),
        in_specs=[
            pl.BlockSpec(
                (D, PADC),
                lambda i, k=k: (0, jnp.minimum(i + k * NB, LAST_BLK)))
            for k in range(4)
        ],
        out_specs=pl.BlockSpec((PADC, 4 * D), lambda i: (i, 0)),
        out_shape=jax.ShapeDtypeStruct((Q, 4 * D), jnp.float32),
    )(table_t, table_t, table_t, table_t)


def kernel(text, embedding_weight):
    text = text.astype(jnp.int32)
    packed = _pack_table(jnp.transpose(embedding_weight))
    embedding_weight = jnp.reshape(packed, (4 * Q, D))
    text = 4 * (text & (Q - 1)) + (text >> 18)
    mesh = plsc.VectorSubcoreMesh(core_axis_name="c", subcore_axis_name="s")
    f = functools.partial(
        pl.kernel,
        mesh=mesh,
        compiler_params=pltpu.CompilerParams(use_tc_tiling_on_sc=False),
        out_type=jax.ShapeDtypeStruct((BATCH, D), jnp.float32),
        scratch_types=[
            pltpu.VMEM((2, S, BPW), jnp.int32),
            pltpu.VMEM((2, S * BPW, D), jnp.float32),
            pltpu.VMEM((S * BPW,), jnp.int32),
            pltpu.VMEM((BPW, D), jnp.float32),
            pltpu.VMEM_SHARED((NS * BPW, D), jnp.float32),
            pltpu.SemaphoreType.DMA,
            pltpu.SemaphoreType.DMA,
        ],
    )(_emb_avg)
    return f(text, embedding_weight)
